# baseline (speedup 1.0000x reference)
import functools

import jax
import jax.numpy as jnp
from jax import lax
from jax.experimental import pallas as pl
from jax.experimental.pallas import tpu as pltpu
from jax.experimental.pallas import tpu_sc as plsc

SEQ = 32768
NE = 64
NEP = 80
NG = 8
GS = NE // NG
NC, NS, L = 2, 16, 16
NW = NC * NS
ROWS_PER_W = SEQ // NW
CR = 256
NCHUNK = ROWS_PER_W // CR
NBLK = CR // L


def _treemax(vals):
    while len(vals) > 1:
        vals = [jnp.maximum(vals[2 * i], vals[2 * i + 1])
                for i in range(len(vals) // 2)]
    return vals[0]


def _treemin(vals):
    while len(vals) > 1:
        vals = [jnp.minimum(vals[2 * i], vals[2 * i + 1])
                for i in range(len(vals) // 2)]
    return vals[0]


def _treesum(vals):
    while len(vals) > 1:
        vals = [vals[2 * i] + vals[2 * i + 1]
                for i in range(len(vals) // 2)]
    return vals[0]


def _router_body(in_hbm, rw_hbm, w_hbm, ids_hbm, cnt_hbm,
                 in_v, rw_v, w_v, ids_v, cnt_v,
                 sem_in0, sem_in1, sem_out0, sem_out1):
    sem_in = [sem_in0, sem_in1]
    sem_out = [sem_out0, sem_out1]
    wid = lax.axis_index("s") * NC + lax.axis_index("c")
    base = wid * ROWS_PER_W

    lanes = jnp.arange(L, dtype=jnp.int32)
    zeros_f = jnp.zeros((L,), jnp.float32)
    ones_f = jnp.ones((L,), jnp.float32)

    for e in range(NE):
        cnt_v[pl.ds(e * L, L)] = zeros_f

    def make_block_body(ibuf):
        in_b = in_v.at[ibuf]
        rw_b = rw_v.at[ibuf]
        w_b = w_v.at[ibuf]
        ids_b = ids_v.at[ibuf]

        def block_body(b):
            r = b * L
            rvec = r + lanes

            gmax = []
            gidx = []
            tg = []
            for g in range(NG):
                x = [plsc.load_gather(
                        in_b, [rvec, jnp.full((L,), g * GS + j, jnp.int32)])
                     for j in range(GS)]
                best = _treemax(list(x))
                cand = [jnp.where(x[j] == best,
                                  jnp.full((L,), j, jnp.int32),
                                  jnp.full((L,), GS, jnp.int32))
                        for j in range(GS)]
                bidx = _treemin(cand)
                q = [jnp.exp(x[j] - best) for j in range(GS)]
                for j in range(GS):
                    plsc.store_scatter(
                        rw_b, [rvec, jnp.full((L,), g * GS + j, jnp.int32)],
                        q[j])
                t = _treesum(q)
                gmax.append(best)
                gidx.append(bidx)
                tg.append(t)

            m = _treemax(list(gmax))
            pg = [jnp.exp(gmax[g] - m) for g in range(NG)]
            tot = _treesum(list(pg))
            tinv = ones_f / tot
            s = _treesum([pg[g] * tg[g] for g in range(NG)])
            sinv = ones_f / s

            for g in range(NG):
                w_b[g, pl.ds(r, L)] = pg[g] * tinv
                gid = gidx[g] + (g * GS)
                ids_b[g, pl.ds(r, L)] = gid
                plsc.addupdate_scatter(cnt_v, [gid * L + lanes], ones_f)
                fct = pg[g] * sinv
                for j in range(GS):
                    cvec = jnp.full((L,), g * GS + j, jnp.int32)
                    qv = plsc.load_gather(rw_b, [rvec, cvec])
                    plsc.store_scatter(rw_b, [rvec, cvec], qv * fct)

        return block_body

    def start_in(c):
        row0 = base + c * CR
        return pltpu.async_copy(in_hbm.at[pl.ds(row0, CR), :],
                                in_v.at[c % 2, :, pl.ds(0, NE)],
                                sem_in[c % 2])

    in_dma = [start_in(0)]
    out_dma = {}
    for c in range(NCHUNK):
        if c + 1 < NCHUNK:
            in_dma.append(start_in(c + 1))
        in_dma[c].wait()
        if c >= 2:
            for h in out_dma[c - 2]:
                h.wait()
        plsc.parallel_loop(0, NBLK, 1, unroll=2)(make_block_body(c % 2))
        row0 = base + c * CR
        out_dma[c] = [
            pltpu.async_copy(rw_v.at[c % 2, :, pl.ds(0, NE)],
                             rw_hbm.at[pl.ds(row0, CR), :],
                             sem_out[c % 2]),
            pltpu.async_copy(w_v.at[c % 2], w_hbm.at[:, pl.ds(row0, CR)],
                             sem_out[c % 2]),
            pltpu.async_copy(ids_v.at[c % 2], ids_hbm.at[:, pl.ds(row0, CR)],
                             sem_out[c % 2]),
        ]
    for c in range(max(0, NCHUNK - 2), NCHUNK):
        for h in out_dma[c]:
            h.wait()

    acc = [jnp.zeros((L,), jnp.float32) for _ in range(NE // L)]
    for e in range(NE):
        v = cnt_v[pl.ds(e * L, L)]
        sv = jnp.full((L,), jnp.sum(v), jnp.float32)
        q, rr = divmod(e, L)
        acc[q] = jnp.where(lanes == rr, sv, acc[q])
    for q in range(NE // L):
        cnt_v[pl.ds(q * L, L)] = acc[q]
    pltpu.sync_copy(cnt_v.at[pl.ds(0, NE)], cnt_hbm.at[pl.ds(wid * NE, NE)])


_router = functools.partial(
    pl.kernel,
    out_type=[
        jax.ShapeDtypeStruct((SEQ, NE), jnp.float32),
        jax.ShapeDtypeStruct((NG, SEQ), jnp.float32),
        jax.ShapeDtypeStruct((NG, SEQ), jnp.int32),
        jax.ShapeDtypeStruct((NW * NE,), jnp.float32),
    ],
    mesh=plsc.VectorSubcoreMesh(core_axis_name="c", subcore_axis_name="s",
                                num_cores=NC, num_subcores=NS),
    compiler_params=pltpu.CompilerParams(needs_layout_passes=False,
                                         use_tc_tiling_on_sc=False),
    scratch_types=[
        pltpu.VMEM((2, CR, NEP), jnp.float32),
        pltpu.VMEM((2, CR, NEP), jnp.float32),
        pltpu.VMEM((2, NG, CR), jnp.float32),
        pltpu.VMEM((2, NG, CR), jnp.int32),
        pltpu.VMEM((NE * L,), jnp.float32),
        pltpu.SemaphoreType.DMA,
        pltpu.SemaphoreType.DMA,
        pltpu.SemaphoreType.DMA,
        pltpu.SemaphoreType.DMA,
    ],
)(_router_body)


@jax.jit
def kernel(logits):
    rw, w_t, ids_t, cnt_part = _router(logits)
    topk_weights = w_t.T
    topk_ids = ids_t.T
    tokens_per_expert = cnt_part.reshape(NW, NE).sum(axis=0)
    return (logits, rw, topk_weights, topk_ids, tokens_per_expert)


# speedup vs baseline: 1.5906x; 1.5906x over previous
import functools

import jax
import jax.numpy as jnp
from jax import lax
from jax.experimental import pallas as pl
from jax.experimental.pallas import tpu as pltpu
from jax.experimental.pallas import tpu_sc as plsc

SEQ = 32768
NE = 64
NG = 8
GS = NE // NG
NC, NS, L = 2, 16, 16
NW = NC * NS
ROWS_PER_W = SEQ // NW
CR = 256
NCHUNK = ROWS_PER_W // CR
NBLK = CR // L
BR = 2048


def _treemax(vals):
    while len(vals) > 1:
        vals = [jnp.maximum(vals[2 * i], vals[2 * i + 1])
                for i in range(len(vals) // 2)]
    return vals[0]


def _treemin(vals):
    while len(vals) > 1:
        vals = [jnp.minimum(vals[2 * i], vals[2 * i + 1])
                for i in range(len(vals) // 2)]
    return vals[0]


def _treesum(vals):
    while len(vals) > 1:
        vals = [vals[2 * i] + vals[2 * i + 1]
                for i in range(len(vals) // 2)]
    return vals[0]


def _softmax_tc_body(x_ref, rw_ref, rwt_ref):
    x = x_ref[...]
    m = jnp.max(x, axis=1, keepdims=True)
    e = jnp.exp(x - m)
    p = e / jnp.sum(e, axis=1, keepdims=True)
    rw_ref[...] = p
    rwt_ref[...] = p.T


_softmax_tc = pl.pallas_call(
    _softmax_tc_body,
    grid=(SEQ // BR,),
    in_specs=[pl.BlockSpec((BR, NE), lambda i: (i, 0))],
    out_specs=[pl.BlockSpec((BR, NE), lambda i: (i, 0)),
               pl.BlockSpec((NE, BR), lambda i: (0, i))],
    out_shape=[jax.ShapeDtypeStruct((SEQ, NE), jnp.float32),
               jax.ShapeDtypeStruct((NE, SEQ), jnp.float32)],
)


def _router_body(p_hbm, w_hbm, ids_hbm, cnt_hbm,
                 in_v, w_v, ids_v, cnt_v,
                 sem_in0, sem_in1, sem_out0, sem_out1):
    sem_in = [sem_in0, sem_in1]
    sem_out = [sem_out0, sem_out1]
    wid = lax.axis_index("s") * NC + lax.axis_index("c")
    base = wid * ROWS_PER_W

    lanes = jnp.arange(L, dtype=jnp.int32)
    zeros_f = jnp.zeros((L,), jnp.float32)
    ones_f = jnp.ones((L,), jnp.float32)

    for e in range(NE):
        cnt_v[pl.ds(e * L, L)] = zeros_f

    def make_block_body(ibuf):
        in_b = in_v.at[ibuf]
        w_b = w_v.at[ibuf]
        ids_b = ids_v.at[ibuf]

        def block_body(b):
            r = b * L

            gmax = []
            gidx = []
            for g in range(NG):
                x = [in_b[g * GS + j, pl.ds(r, L)] for j in range(GS)]
                best = _treemax(list(x))
                cand = [jnp.where(x[j] == best,
                                  jnp.full((L,), j, jnp.int32),
                                  jnp.full((L,), GS, jnp.int32))
                        for j in range(GS)]
                gmax.append(best)
                gidx.append(_treemin(cand))

            tot = _treesum(list(gmax))
            tinv = ones_f / tot

            for g in range(NG):
                w_b[g, pl.ds(r, L)] = gmax[g] * tinv
                gid = gidx[g] + (g * GS)
                ids_b[g, pl.ds(r, L)] = gid
                plsc.addupdate_scatter(cnt_v, [gid * L + lanes], ones_f)

        return block_body

    def start_in(c):
        row0 = base + c * CR
        return pltpu.async_copy(p_hbm.at[:, pl.ds(row0, CR)],
                                in_v.at[c % 2], sem_in[c % 2])

    in_dma = [start_in(0)]
    out_dma = {}
    for c in range(NCHUNK):
        if c + 1 < NCHUNK:
            in_dma.append(start_in(c + 1))
        in_dma[c].wait()
        if c >= 2:
            for h in out_dma[c - 2]:
                h.wait()
        plsc.parallel_loop(0, NBLK, 1, unroll=2)(make_block_body(c % 2))
        row0 = base + c * CR
        out_dma[c] = [
            pltpu.async_copy(w_v.at[c % 2], w_hbm.at[:, pl.ds(row0, CR)],
                             sem_out[c % 2]),
            pltpu.async_copy(ids_v.at[c % 2], ids_hbm.at[:, pl.ds(row0, CR)],
                             sem_out[c % 2]),
        ]
    for c in range(max(0, NCHUNK - 2), NCHUNK):
        for h in out_dma[c]:
            h.wait()

    acc = [jnp.zeros((L,), jnp.float32) for _ in range(NE // L)]
    for e in range(NE):
        v = cnt_v[pl.ds(e * L, L)]
        sv = jnp.full((L,), jnp.sum(v), jnp.float32)
        q, rr = divmod(e, L)
        acc[q] = jnp.where(lanes == rr, sv, acc[q])
    for q in range(NE // L):
        cnt_v[pl.ds(q * L, L)] = acc[q]
    pltpu.sync_copy(cnt_v.at[pl.ds(0, NE)], cnt_hbm.at[pl.ds(wid * NE, NE)])


_router = functools.partial(
    pl.kernel,
    out_type=[
        jax.ShapeDtypeStruct((NG, SEQ), jnp.float32),
        jax.ShapeDtypeStruct((NG, SEQ), jnp.int32),
        jax.ShapeDtypeStruct((NW * NE,), jnp.float32),
    ],
    mesh=plsc.VectorSubcoreMesh(core_axis_name="c", subcore_axis_name="s",
                                num_cores=NC, num_subcores=NS),
    compiler_params=pltpu.CompilerParams(needs_layout_passes=False),
    scratch_types=[
        pltpu.VMEM((2, NE, CR), jnp.float32),
        pltpu.VMEM((2, NG, CR), jnp.float32),
        pltpu.VMEM((2, NG, CR), jnp.int32),
        pltpu.VMEM((NE * L,), jnp.float32),
        pltpu.SemaphoreType.DMA,
        pltpu.SemaphoreType.DMA,
        pltpu.SemaphoreType.DMA,
        pltpu.SemaphoreType.DMA,
    ],
)(_router_body)


@jax.jit
def kernel(logits):
    rw, rwt = _softmax_tc(logits)
    w_t, ids_t, cnt_part = _router(rwt)
    topk_weights = w_t.T
    topk_ids = ids_t.T
    tokens_per_expert = cnt_part.reshape(NW, NE).sum(axis=0)
    return (logits, rw, topk_weights, topk_ids, tokens_per_expert)
